# baseline (speedup 1.0000x reference)
import functools

import jax
import jax.numpy as jnp
from jax import lax
from jax.experimental import pallas as pl
from jax.experimental.pallas import tpu as pltpu
from jax.experimental.pallas import tpu_sc as plsc

_NC = 2
_NS = 16
_NW = _NC * _NS
_L = 16


def _build_gather(B, V, J, T, C):
    assert B % _NW == 0
    bpw = B // _NW
    tw = T * C
    n_idx = bpw * tw
    chunk = 96
    n_chunks = n_idx // chunk
    assert n_idx % chunk == 0 and chunk % _L == 0
    vw = V * C

    mesh = plsc.VectorSubcoreMesh(
        core_axis_name="c", subcore_axis_name="s",
        num_cores=_NC, num_subcores=_NS)

    @functools.partial(
        pl.kernel,
        out_type=jax.ShapeDtypeStruct((B * tw,), jnp.float32),
        mesh=mesh,
        compiler_params=pltpu.CompilerParams(
            needs_layout_passes=False, use_tc_tiling_on_sc=False),
        scratch_types=[
            pltpu.VMEM((T,), jnp.int32),
            pltpu.VMEM((n_chunks, chunk), jnp.int32),
            pltpu.VMEM((n_idx,), jnp.float32),
            pltpu.SemaphoreType.DMA,
        ],
    )
    def gather_call(vert_flat, eidx_hbm, tips_out,
                    eidx_v, gidx_v, rows_v, gsem):
        wid = lax.axis_index("s") * _NC + lax.axis_index("c")
        b0 = wid * bpw

        pltpu.sync_copy(eidx_hbm, eidx_v)
        lane = lax.iota(jnp.int32, _L)

        for v in range(n_idx // _L):
            k = v * _L + lane
            b_local = lax.div(k, jnp.int32(tw))
            r = k - b_local * jnp.int32(tw)
            t = lax.div(r, jnp.int32(C))
            c = r - t * jnp.int32(C)
            gid = ((b0 + b_local) * jnp.int32(vw)
                   + plsc.load_gather(eidx_v, [t]) * jnp.int32(C) + c)
            gidx_v[(v * _L) // chunk, pl.ds((v * _L) % chunk, _L)] = gid

        gcps = [
            pltpu.async_copy(
                vert_flat.at[gidx_v.at[ch]],
                rows_v.at[pl.ds(ch * chunk, chunk)], gsem)
            for ch in range(n_chunks)
        ]
        for g in gcps:
            g.wait()

        pltpu.sync_copy(rows_v, tips_out.at[pl.ds(b0 * tw, n_idx)])

    return gather_call


def _assemble_body(J, T, C, joints_ref, tips_ref, out_ref):
    jw = J * C
    out_ref[:, :jw] = joints_ref[...]
    out_ref[:, jw:] = tips_ref[...]


def _build_assemble(B, J, T, C, G=256):
    jw = J * C
    tw = T * C
    body = functools.partial(_assemble_body, J, T, C)
    return pl.pallas_call(
        body,
        grid=(B // G,),
        in_specs=[
            pl.BlockSpec((G, jw), lambda i: (i, 0)),
            pl.BlockSpec((G, tw), lambda i: (i, 0)),
        ],
        out_specs=pl.BlockSpec((G, jw + tw), lambda i: (i, 0)),
        out_shape=jax.ShapeDtypeStruct((B, jw + tw), jnp.float32),
    )


def kernel(vertices, joints, extra_joints_idxs):
    B, V, C = vertices.shape
    J = joints.shape[1]
    T = extra_joints_idxs.shape[0]
    vert_flat = vertices.reshape(B * V * C)
    eidx = extra_joints_idxs.astype(jnp.int32)
    tips = _build_gather(B, V, J, T, C)(vert_flat, eidx)
    joints2 = joints.reshape(B, J * C)
    tips2 = tips.reshape(B, T * C)
    out2 = _build_assemble(B, J, T, C)(joints2, tips2)
    return out2.reshape(B, J + T, C)


# speedup vs baseline: 15.2651x; 15.2651x over previous
import functools

import jax
import jax.numpy as jnp
from jax.experimental import pallas as pl
from jax.experimental.pallas import tpu as pltpu


def _body(J, T, C, sref, *refs):
    joints_ref = refs[T]
    out_ref = refs[T + 1]
    out_ref[:, : J * C] = joints_ref[...]
    for t in range(T):
        r = sref[t] % 8
        sliver = refs[t][:, pl.ds(r, 1), :]
        out_ref[:, J * C + t * C : J * C + (t + 1) * C] = sliver[:, 0, :]


def _build_call(B, V, J, T, C, G=256):
    jw = J * C

    def vert_spec(t):
        return pl.BlockSpec(
            (G, 8, C), lambda b, sref, t=t: (b, sref[t] // 8, 0))

    grid_spec = pltpu.PrefetchScalarGridSpec(
        num_scalar_prefetch=1,
        grid=(B // G,),
        in_specs=(
            [vert_spec(t) for t in range(T)]
            + [pl.BlockSpec((G, jw), lambda b, sref: (b, 0))]
        ),
        out_specs=pl.BlockSpec((G, jw + T * C), lambda b, sref: (b, 0)),
    )
    return pl.pallas_call(
        functools.partial(_body, J, T, C),
        grid_spec=grid_spec,
        out_shape=jax.ShapeDtypeStruct((B, jw + T * C), jnp.float32),
    )


def kernel(vertices, joints, extra_joints_idxs):
    B, V, C = vertices.shape
    J = joints.shape[1]
    T = extra_joints_idxs.shape[0]
    eidx = extra_joints_idxs.astype(jnp.int32)
    joints2 = joints.reshape(B, J * C)
    call = _build_call(B, V, J, T, C)
    out2 = call(eidx, *([vertices] * T), joints2)
    return out2.reshape(B, J + T, C)


# speedup vs baseline: 22786.6888x; 1492.7300x over previous
import functools

import jax
import jax.numpy as jnp
from jax.experimental import pallas as pl
from jax.experimental.pallas import tpu as pltpu


def _body(J, T, C, sref, *refs):
    joints_ref = refs[T]
    out_ref = refs[T + 1]
    out_ref[:, :J, :] = joints_ref[...]
    for t in range(T):
        r = sref[t] % 8
        out_ref[:, J + t, :] = refs[t][:, pl.ds(r, 1), :][:, 0, :]


def _build_call(B, V, J, T, C):
    def vert_spec(t):
        return pl.BlockSpec(
            (C, 8, B), lambda i, sref, t=t: (0, sref[t] // 8, 0))

    grid_spec = pltpu.PrefetchScalarGridSpec(
        num_scalar_prefetch=1,
        grid=(1,),
        in_specs=(
            [vert_spec(t) for t in range(T)]
            + [pl.BlockSpec((C, J, B), lambda i, sref: (0, 0, 0))]
        ),
        out_specs=pl.BlockSpec((C, J + T, B), lambda i, sref: (0, 0, 0)),
    )
    return pl.pallas_call(
        functools.partial(_body, J, T, C),
        grid_spec=grid_spec,
        out_shape=jax.ShapeDtypeStruct((C, J + T, B), jnp.float32),
    )


def kernel(vertices, joints, extra_joints_idxs):
    B, V, C = vertices.shape
    J = joints.shape[1]
    T = extra_joints_idxs.shape[0]
    eidx = extra_joints_idxs.astype(jnp.int32)
    vt = vertices.transpose(2, 1, 0)
    jt = joints.transpose(2, 1, 0)
    out_t = _build_call(B, V, J, T, C)(eidx, *([vt] * T), jt)
    return out_t.transpose(2, 1, 0)
